# baseline (speedup 1.0000x reference)
import functools

import jax
import jax.numpy as jnp
from jax.experimental import pallas as pl
from jax.experimental.pallas import tpu as pltpu

B, S, D, H, DFF, E = 4, 2048, 1024, 16, 4096, 8
DH = D // H
EPS = 1e-12

_BF = jnp.bfloat16
_F32 = jnp.float32


def _qkv_body(x_ref, w_ref, b_ref, o_ref):
    cr = 256
    for ci in range(o_ref.shape[0] // cr):
        sl = slice(ci * cr, (ci + 1) * cr)
        acc = jax.lax.dot_general(
            x_ref[sl, :], w_ref[...], (((1,), (0,)), ((), ())),
            preferred_element_type=_F32)
        o_ref[sl, :] = (acc + b_ref[...]).astype(_BF)


def _qkv_proj(x_bf, w_bf, bias):
    rows = B * S
    br = 512
    return pl.pallas_call(
        _qkv_body,
        grid=(rows // br,),
        in_specs=[
            pl.BlockSpec((br, D), lambda i: (i, 0)),
            pl.BlockSpec((D, 3 * D), lambda i: (0, 0)),
            pl.BlockSpec((1, 3 * D), lambda i: (0, 0)),
        ],
        out_specs=pl.BlockSpec((br, 3 * D), lambda i: (i, 0)),
        out_shape=jax.ShapeDtypeStruct((rows, 3 * D), _BF),
    )(x_bf, w_bf, bias)


def _attn_body(q_ref, kt_ref, v_ref, o_ref):
    kt = kt_ref[0, 0]
    v = v_ref[0, 0]
    cr = 1024
    for ci in range(o_ref.shape[2] // cr):
        q = q_ref[0, 0, ci * cr:(ci + 1) * cr, :]
        s = jax.lax.dot_general(
            q, kt, (((1,), (0,)), ((), ())),
            preferred_element_type=_F32).astype(_BF)
        m = jnp.max(s, axis=1, keepdims=True)
        e = jnp.exp(s - m)
        ctx = jax.lax.dot_general(
            e, v, (((1,), (0,)), ((), ())),
            preferred_element_type=_F32)
        l = ctx[:, DH:DH + 1]
        o_ref[0, 0, ci * cr:(ci + 1) * cr, :] = (
            ctx[:, :DH] * (1.0 / l)).astype(_BF)


def _attention(q, kt, v_aug):
    bq = 2048
    return pl.pallas_call(
        _attn_body,
        grid=(B, H, S // bq),
        in_specs=[
            pl.BlockSpec((1, 1, bq, DH), lambda b, h, i: (b, h, i, 0)),
            pl.BlockSpec((1, 1, DH, S), lambda b, h, i: (b, h, 0, 0)),
            pl.BlockSpec((1, 1, S, 2 * DH), lambda b, h, i: (b, h, 0, 0)),
        ],
        out_specs=pl.BlockSpec((1, 1, bq, DH), lambda b, h, i: (b, h, i, 0)),
        out_shape=jax.ShapeDtypeStruct((B, H, S, DH), _BF),
        compiler_params=pltpu.CompilerParams(
            dimension_semantics=("parallel", "parallel", "parallel")),
    )(q, kt, v_aug)


def _ln(y, g, b):
    mu = jnp.mean(y, axis=1, keepdims=True)
    yc = y - mu
    var = jnp.mean(yc * yc, axis=1, keepdims=True)
    return yc * jax.lax.rsqrt(var + EPS) * g + b


def _post_body(nblk_per_b, ctx_ref, wo_ref, bo_ref, x_ref,
               g1_ref, b1_ref, g2_ref, b2_ref,
               attn_ref, ln2_ref, sum_ref):
    i = pl.program_id(0)
    y = jax.lax.dot_general(
        ctx_ref[...], wo_ref[...], (((1,), (0,)), ((), ())),
        preferred_element_type=_F32)
    y = y + bo_ref[...] + x_ref[...]
    attn = _ln(y, g1_ref[...], b1_ref[...])
    attn_ref[...] = attn
    ln2 = _ln(attn, g2_ref[...], b2_ref[...])
    ln2_ref[...] = ln2.astype(_BF)
    psum = jnp.sum(ln2, axis=0)[None, None, :]

    @pl.when(i % nblk_per_b == 0)
    def _init():
        sum_ref[...] = psum

    @pl.when(i % nblk_per_b != 0)
    def _acc():
        sum_ref[...] += psum


def _post_attn(ctx_bf, wo_bf, bo, x2, g1, b1, g2, b2):
    rows = B * S
    br = 256
    nblk_per_b = S // br
    return pl.pallas_call(
        functools.partial(_post_body, nblk_per_b),
        grid=(rows // br,),
        in_specs=[
            pl.BlockSpec((br, D), lambda i: (i, 0)),
            pl.BlockSpec((D, D), lambda i: (0, 0)),
            pl.BlockSpec((1, D), lambda i: (0, 0)),
            pl.BlockSpec((br, D), lambda i: (i, 0)),
            pl.BlockSpec((1, D), lambda i: (0, 0)),
            pl.BlockSpec((1, D), lambda i: (0, 0)),
            pl.BlockSpec((1, D), lambda i: (0, 0)),
            pl.BlockSpec((1, D), lambda i: (0, 0)),
        ],
        out_specs=[
            pl.BlockSpec((br, D), lambda i: (i, 0)),
            pl.BlockSpec((br, D), lambda i: (i, 0)),
            pl.BlockSpec((1, 1, D), lambda i: (i // nblk_per_b, 0, 0)),
        ],
        out_shape=[
            jax.ShapeDtypeStruct((rows, D), _F32),
            jax.ShapeDtypeStruct((rows, D), _BF),
            jax.ShapeDtypeStruct((B, 1, D), _F32),
        ],
        compiler_params=pltpu.CompilerParams(
            dimension_semantics=("arbitrary",)),
    )(ctx_bf, wo_bf, bo, x2, g1, b1, g2, b2)


def _gate_body(sum_ref, gw_ref, logit_ref):
    logit_ref[...] = jax.lax.dot_general(
        sum_ref[...] * (1.0 / S), gw_ref[...], (((1,), (0,)), ((), ())),
        preferred_element_type=_F32,
        precision=jax.lax.Precision.HIGHEST)


def _gate_logits(sum_ln, gate_w):
    return pl.pallas_call(
        _gate_body,
        in_specs=[
            pl.BlockSpec((B, D), lambda: (0, 0)),
            pl.BlockSpec((D, E), lambda: (0, 0)),
        ],
        out_specs=pl.BlockSpec((B, E), lambda: (0, 0)),
        out_shape=jax.ShapeDtypeStruct((B, E), _F32),
    )(sum_ln, gate_w)


def _gelu_exact(x):
    return 0.5 * x * (1.0 + jax.lax.erf(x * 0.7071067811865476))


def _ffn_body(nf, choice_ref, ln_ref, wu_ref, bu_ref, wd_ref, bd_ref,
              out_ref):
    f = pl.program_id(2)
    wu = wu_ref[0].astype(_BF)
    wd = wd_ref[0].astype(_BF)
    cr = 256
    nc = out_ref.shape[1] // cr
    os = []
    for ci in range(nc):
        sl = slice(ci * cr, (ci + 1) * cr)
        xb = ln_ref[0, sl, :]
        h = jax.lax.dot_general(
            xb, wu, (((1,), (0,)), ((), ())),
            preferred_element_type=_F32) + bu_ref[0]
        h = _gelu_exact(h.astype(_BF))
        os.append(jax.lax.dot_general(
            h, wd, (((1,), (0,)), ((), ())),
            preferred_element_type=_F32))

    for ci in range(nc):
        sl = slice(ci * cr, (ci + 1) * cr)
        o = os[ci]

        @pl.when(f == 0)
        def _init(sl=sl, o=o):
            out_ref[0, sl, :] = o

        @pl.when((f > 0) & (f < nf - 1))
        def _acc(sl=sl, o=o):
            out_ref[0, sl, :] = out_ref[0, sl, :] + o

        @pl.when(f == nf - 1)
        def _fini(sl=sl, o=o):
            out_ref[0, sl, :] = out_ref[0, sl, :] + o + bd_ref[0]


def _moe_ffn(choice, ln3, w_up, b_up3, w_down, b_down3):
    bs = 1024
    fb = 2048
    nf = DFF // fb
    grid = (B, S // bs, nf)
    return pl.pallas_call(
        functools.partial(_ffn_body, nf),
        grid_spec=pltpu.PrefetchScalarGridSpec(
            num_scalar_prefetch=1,
            grid=grid,
            in_specs=[
                pl.BlockSpec((1, bs, D), lambda b, s, f, c: (b, s, 0)),
                pl.BlockSpec((1, D, fb), lambda b, s, f, c: (c[b], 0, f)),
                pl.BlockSpec((1, 1, fb), lambda b, s, f, c: (c[b], 0, f)),
                pl.BlockSpec((1, fb, D), lambda b, s, f, c: (c[b], f, 0)),
                pl.BlockSpec((1, 1, D), lambda b, s, f, c: (c[b], 0, 0)),
            ],
            out_specs=pl.BlockSpec((1, bs, D), lambda b, s, f, c: (b, s, 0)),
        ),
        out_shape=jax.ShapeDtypeStruct((B, S, D), _F32),
        compiler_params=pltpu.CompilerParams(
            dimension_semantics=("arbitrary", "arbitrary", "arbitrary")),
    )(choice, ln3, w_up, b_up3, w_down, b_down3)


def kernel(hidden_states, Wq, bq, Wk, bk, Wv, bv, Wo, bo,
           ln_attn_g, ln_attn_b, ln_moe_g, ln_moe_b,
           gate_W, W_up, b_up, W_down, b_down):
    x2 = hidden_states.reshape(B * S, D)
    x_bf = x2.astype(_BF)

    wqkv = jnp.concatenate([Wq, Wk, Wv], axis=1).astype(_BF)
    bqkv = jnp.concatenate([bq, bk, bv])[None, :]
    qkv = _qkv_proj(x_bf, wqkv, bqkv)

    qkv4 = qkv.reshape(B, S, 3, H, DH)
    q = (qkv4[:, :, 0] * jnp.bfloat16(0.125)).transpose(0, 2, 1, 3)
    kt = qkv4[:, :, 1].transpose(0, 2, 3, 1)
    v = qkv4[:, :, 2].transpose(0, 2, 1, 3)
    ones = jnp.ones((B, H, S, 1), _BF)
    v_aug = jnp.concatenate([v, jnp.broadcast_to(ones, (B, H, S, DH))],
                            axis=-1)
    ctx = _attention(q, kt, v_aug)
    ctx2 = ctx.transpose(0, 2, 1, 3).reshape(B * S, D)

    attn2, ln2, sum_ln = _post_attn(
        ctx2, Wo.astype(_BF), bo[None, :], x2,
        ln_attn_g[None, :], ln_attn_b[None, :],
        ln_moe_g[None, :], ln_moe_b[None, :])
    router_logits = _gate_logits(sum_ln.reshape(B, D), gate_W)
    choice = jnp.argmax(router_logits, axis=-1).astype(jnp.int32)

    moe = _moe_ffn(choice,
                   ln2.reshape(B, S, D),
                   W_up, b_up.reshape(E, 1, DFF),
                   W_down, b_down.reshape(E, 1, D))
    out = moe + attn2.reshape(B, S, D)
    return (out, router_logits)


# speedup vs baseline: 1.0810x; 1.0810x over previous
import functools

import jax
import jax.numpy as jnp
from jax.experimental import pallas as pl
from jax.experimental.pallas import tpu as pltpu

B, S, D, H, DFF, E = 4, 2048, 1024, 16, 4096, 8
DH = D // H
EPS = 1e-12

_BF = jnp.bfloat16
_F32 = jnp.float32


def _qkv_body(x_ref, w_ref, b_ref, o_ref):
    cr = 256
    for ci in range(o_ref.shape[0] // cr):
        sl = slice(ci * cr, (ci + 1) * cr)
        acc = jax.lax.dot_general(
            x_ref[sl, :], w_ref[...], (((1,), (0,)), ((), ())),
            preferred_element_type=_F32)
        o_ref[sl, :] = (acc + b_ref[...]).astype(_BF)


def _qkv_proj(x_bf, w_bf, bias):
    rows = B * S
    br = 512
    return pl.pallas_call(
        _qkv_body,
        grid=(rows // br,),
        in_specs=[
            pl.BlockSpec((br, D), lambda i: (i, 0)),
            pl.BlockSpec((D, 3 * D), lambda i: (0, 0)),
            pl.BlockSpec((1, 3 * D), lambda i: (0, 0)),
        ],
        out_specs=pl.BlockSpec((br, 3 * D), lambda i: (i, 0)),
        out_shape=jax.ShapeDtypeStruct((rows, 3 * D), _BF),
    )(x_bf, w_bf, bias)


def _attn_body(q_ref, kt_ref, v_ref, o_ref):
    kt = kt_ref[0, 0]
    v = v_ref[0, 0]
    cr = 512
    for ci in range(o_ref.shape[2] // cr):
        q = q_ref[0, 0, ci * cr:(ci + 1) * cr, :]
        s = jax.lax.dot_general(
            q, kt, (((1,), (0,)), ((), ())),
            preferred_element_type=_F32).astype(_BF)
        m = jnp.max(s, axis=1, keepdims=True)
        e = jnp.exp(s - m)
        ctx = jax.lax.dot_general(
            e, v, (((1,), (0,)), ((), ())),
            preferred_element_type=_F32)
        l = ctx[:, DH:DH + 1]
        o_ref[0, 0, ci * cr:(ci + 1) * cr, :] = (
            ctx[:, :DH] * (1.0 / l)).astype(_BF)


def _attention(q, kt, v_aug):
    bq = 2048
    return pl.pallas_call(
        _attn_body,
        grid=(B, H, S // bq),
        in_specs=[
            pl.BlockSpec((1, 1, bq, DH), lambda b, h, i: (b, h, i, 0)),
            pl.BlockSpec((1, 1, DH, S), lambda b, h, i: (b, h, 0, 0)),
            pl.BlockSpec((1, 1, S, 2 * DH), lambda b, h, i: (b, h, 0, 0)),
        ],
        out_specs=pl.BlockSpec((1, 1, bq, DH), lambda b, h, i: (b, h, i, 0)),
        out_shape=jax.ShapeDtypeStruct((B, H, S, DH), _BF),
        compiler_params=pltpu.CompilerParams(
            dimension_semantics=("parallel", "parallel", "parallel")),
    )(q, kt, v_aug)


def _ln(y, g, b):
    mu = jnp.mean(y, axis=1, keepdims=True)
    yc = y - mu
    var = jnp.mean(yc * yc, axis=1, keepdims=True)
    return yc * jax.lax.rsqrt(var + EPS) * g + b


def _post_body(nblk_per_b, ctx_ref, wo_ref, bo_ref, x_ref,
               g1_ref, b1_ref, g2_ref, b2_ref,
               attn_ref, ln2_ref, sum_ref):
    i = pl.program_id(0)
    cr = 256
    psums = []
    for ci in range(ctx_ref.shape[0] // cr):
        sl = slice(ci * cr, (ci + 1) * cr)
        y = jax.lax.dot_general(
            ctx_ref[sl, :], wo_ref[...], (((1,), (0,)), ((), ())),
            preferred_element_type=_F32)
        y = y + bo_ref[...] + x_ref[sl, :]
        attn = _ln(y, g1_ref[...], b1_ref[...])
        attn_ref[sl, :] = attn
        ln2 = _ln(attn, g2_ref[...], b2_ref[...])
        ln2_ref[sl, :] = ln2.astype(_BF)
        psums.append(jnp.sum(ln2, axis=0))
    psum = sum(psums)[None, None, :]

    @pl.when(i % nblk_per_b == 0)
    def _init():
        sum_ref[...] = psum

    @pl.when(i % nblk_per_b != 0)
    def _acc():
        sum_ref[...] += psum


def _post_attn(ctx_bf, wo_bf, bo, x2, g1, b1, g2, b2):
    rows = B * S
    br = 512
    nblk_per_b = S // br
    return pl.pallas_call(
        functools.partial(_post_body, nblk_per_b),
        grid=(rows // br,),
        in_specs=[
            pl.BlockSpec((br, D), lambda i: (i, 0)),
            pl.BlockSpec((D, D), lambda i: (0, 0)),
            pl.BlockSpec((1, D), lambda i: (0, 0)),
            pl.BlockSpec((br, D), lambda i: (i, 0)),
            pl.BlockSpec((1, D), lambda i: (0, 0)),
            pl.BlockSpec((1, D), lambda i: (0, 0)),
            pl.BlockSpec((1, D), lambda i: (0, 0)),
            pl.BlockSpec((1, D), lambda i: (0, 0)),
        ],
        out_specs=[
            pl.BlockSpec((br, D), lambda i: (i, 0)),
            pl.BlockSpec((br, D), lambda i: (i, 0)),
            pl.BlockSpec((1, 1, D), lambda i: (i // nblk_per_b, 0, 0)),
        ],
        out_shape=[
            jax.ShapeDtypeStruct((rows, D), _F32),
            jax.ShapeDtypeStruct((rows, D), _BF),
            jax.ShapeDtypeStruct((B, 1, D), _F32),
        ],
        compiler_params=pltpu.CompilerParams(
            dimension_semantics=("arbitrary",)),
    )(ctx_bf, wo_bf, bo, x2, g1, b1, g2, b2)


def _gate_body(sum_ref, gw_ref, logit_ref):
    logit_ref[...] = jax.lax.dot_general(
        sum_ref[...] * (1.0 / S), gw_ref[...], (((1,), (0,)), ((), ())),
        preferred_element_type=_F32,
        precision=jax.lax.Precision.HIGHEST)


def _gate_logits(sum_ln, gate_w):
    return pl.pallas_call(
        _gate_body,
        in_specs=[
            pl.BlockSpec((B, D), lambda: (0, 0)),
            pl.BlockSpec((D, E), lambda: (0, 0)),
        ],
        out_specs=pl.BlockSpec((B, E), lambda: (0, 0)),
        out_shape=jax.ShapeDtypeStruct((B, E), _F32),
    )(sum_ln, gate_w)


def _gelu_exact(x):
    return 0.5 * x * (1.0 + jax.lax.erf(x * 0.7071067811865476))


def _ffn_body(nf, choice_ref, ln_ref, wu_ref, bu_ref, wd_ref, bd_ref,
              out_ref):
    f = pl.program_id(2)
    wu = wu_ref[0].astype(_BF)
    wd = wd_ref[0].astype(_BF)
    cr = 256
    nc = out_ref.shape[1] // cr
    os = []
    for ci in range(nc):
        sl = slice(ci * cr, (ci + 1) * cr)
        xb = ln_ref[0, sl, :]
        h = jax.lax.dot_general(
            xb, wu, (((1,), (0,)), ((), ())),
            preferred_element_type=_F32) + bu_ref[0]
        h = _gelu_exact(h.astype(_BF))
        os.append(jax.lax.dot_general(
            h, wd, (((1,), (0,)), ((), ())),
            preferred_element_type=_F32))

    for ci in range(nc):
        sl = slice(ci * cr, (ci + 1) * cr)
        o = os[ci]

        @pl.when(f == 0)
        def _init(sl=sl, o=o):
            out_ref[0, sl, :] = o

        @pl.when((f > 0) & (f < nf - 1))
        def _acc(sl=sl, o=o):
            out_ref[0, sl, :] = out_ref[0, sl, :] + o

        @pl.when(f == nf - 1)
        def _fini(sl=sl, o=o):
            out_ref[0, sl, :] = out_ref[0, sl, :] + o + bd_ref[0]


def _moe_ffn(choice, ln3, w_up, b_up3, w_down, b_down3):
    bs = 1024
    fb = 2048
    nf = DFF // fb
    grid = (B, S // bs, nf)
    return pl.pallas_call(
        functools.partial(_ffn_body, nf),
        grid_spec=pltpu.PrefetchScalarGridSpec(
            num_scalar_prefetch=1,
            grid=grid,
            in_specs=[
                pl.BlockSpec((1, bs, D), lambda b, s, f, c: (b, s, 0)),
                pl.BlockSpec((1, D, fb), lambda b, s, f, c: (c[b], 0, f)),
                pl.BlockSpec((1, 1, fb), lambda b, s, f, c: (c[b], 0, f)),
                pl.BlockSpec((1, fb, D), lambda b, s, f, c: (c[b], f, 0)),
                pl.BlockSpec((1, 1, D), lambda b, s, f, c: (c[b], 0, 0)),
            ],
            out_specs=pl.BlockSpec((1, bs, D), lambda b, s, f, c: (b, s, 0)),
        ),
        out_shape=jax.ShapeDtypeStruct((B, S, D), _F32),
        compiler_params=pltpu.CompilerParams(
            dimension_semantics=("arbitrary", "arbitrary", "arbitrary")),
    )(choice, ln3, w_up, b_up3, w_down, b_down3)


def kernel(hidden_states, Wq, bq, Wk, bk, Wv, bv, Wo, bo,
           ln_attn_g, ln_attn_b, ln_moe_g, ln_moe_b,
           gate_W, W_up, b_up, W_down, b_down):
    x2 = hidden_states.reshape(B * S, D)
    x_bf = x2.astype(_BF)

    wqkv = jnp.concatenate([Wq * 0.125, Wk, Wv], axis=1).astype(_BF)
    bqkv = jnp.concatenate([bq * 0.125, bk, bv])[None, :]
    qkv = _qkv_proj(x_bf, wqkv, bqkv)

    qkv4 = qkv.reshape(B, S, 3, H, DH)
    q = qkv4[:, :, 0].transpose(0, 2, 1, 3)
    kt = qkv4[:, :, 1].transpose(0, 2, 3, 1)
    v = qkv4[:, :, 2].transpose(0, 2, 1, 3)
    ones = jnp.ones((B, H, S, 1), _BF)
    v_aug = jnp.concatenate([v, jnp.broadcast_to(ones, (B, H, S, DH))],
                            axis=-1)
    ctx = _attention(q, kt, v_aug)
    ctx2 = ctx.transpose(0, 2, 1, 3).reshape(B * S, D)

    attn2, ln2, sum_ln = _post_attn(
        ctx2, Wo.astype(_BF), bo[None, :], x2,
        ln_attn_g[None, :], ln_attn_b[None, :],
        ln_moe_g[None, :], ln_moe_b[None, :])
    router_logits = _gate_logits(sum_ln.reshape(B, D), gate_W)
    choice = jnp.argmax(router_logits, axis=-1).astype(jnp.int32)

    moe = _moe_ffn(choice,
                   ln2.reshape(B, S, D),
                   W_up, b_up.reshape(E, 1, DFF),
                   W_down, b_down.reshape(E, 1, D))
    out = moe + attn2.reshape(B, S, D)
    return (out, router_logits)


# speedup vs baseline: 1.1621x; 1.0750x over previous
import functools

import jax
import jax.numpy as jnp
from jax.experimental import pallas as pl
from jax.experimental.pallas import tpu as pltpu

B, S, D, H, DFF, E = 4, 2048, 1024, 16, 4096, 8
DH = D // H
EPS = 1e-12

_BF = jnp.bfloat16
_F32 = jnp.float32


def _qkv_body(x_ref, w_ref, b_ref, o_ref):
    cr = 256
    for ci in range(o_ref.shape[0] // cr):
        sl = slice(ci * cr, (ci + 1) * cr)
        acc = jax.lax.dot_general(
            x_ref[sl, :], w_ref[...], (((1,), (0,)), ((), ())),
            preferred_element_type=_F32)
        o_ref[sl, :] = (acc + b_ref[...]).astype(_BF)


def _qkv_proj(x_bf, w_bf, bias):
    rows = B * S
    br = 512
    return pl.pallas_call(
        _qkv_body,
        grid=(rows // br,),
        in_specs=[
            pl.BlockSpec((br, D), lambda i: (i, 0)),
            pl.BlockSpec((D, 3 * D), lambda i: (0, 0)),
            pl.BlockSpec((1, 3 * D), lambda i: (0, 0)),
        ],
        out_specs=pl.BlockSpec((br, 3 * D), lambda i: (i, 0)),
        out_shape=jax.ShapeDtypeStruct((rows, 3 * D), _BF),
    )(x_bf, w_bf, bias)


def _attn_body(q_ref, kt_ref, v_ref, o_ref):
    kt = kt_ref[0, 0]
    v = v_ref[0, 0]
    cr = 512
    nc = o_ref.shape[2] // cr
    es = []
    for ci in range(nc):
        q = q_ref[0, 0, ci * cr:(ci + 1) * cr, :]
        s = jax.lax.dot_general(
            q, kt, (((1,), (0,)), ((), ())),
            preferred_element_type=_F32).astype(_BF)
        m = jnp.max(s, axis=1, keepdims=True)
        es.append(jnp.exp(s - m))
    for ci in range(nc):
        ctx = jax.lax.dot_general(
            es[ci], v, (((1,), (0,)), ((), ())),
            preferred_element_type=_F32)
        l = ctx[:, DH:DH + 1]
        o_ref[0, 0, ci * cr:(ci + 1) * cr, :] = (
            ctx[:, :DH] * (1.0 / l)).astype(_BF)


def _attention(q, kt, v_aug):
    bq = 2048
    return pl.pallas_call(
        _attn_body,
        grid=(B, H, S // bq),
        in_specs=[
            pl.BlockSpec((1, 1, bq, DH), lambda b, h, i: (b, h, i, 0)),
            pl.BlockSpec((1, 1, DH, S), lambda b, h, i: (b, h, 0, 0)),
            pl.BlockSpec((1, 1, S, 2 * DH), lambda b, h, i: (b, h, 0, 0)),
        ],
        out_specs=pl.BlockSpec((1, 1, bq, DH), lambda b, h, i: (b, h, i, 0)),
        out_shape=jax.ShapeDtypeStruct((B, H, S, DH), _BF),
        compiler_params=pltpu.CompilerParams(
            dimension_semantics=("parallel", "parallel", "parallel")),
    )(q, kt, v_aug)


def _ln(y, g, b):
    mu = jnp.mean(y, axis=1, keepdims=True)
    yc = y - mu
    var = jnp.mean(yc * yc, axis=1, keepdims=True)
    return yc * jax.lax.rsqrt(var + EPS) * g + b


def _post_body(nblk_per_b, ctx_ref, wo_ref, bo_ref, x_ref,
               g1_ref, b1_ref, g2_ref, b2_ref,
               attn_ref, ln2_ref, sum_ref):
    i = pl.program_id(0)
    cr = 256
    psums = []
    for ci in range(ctx_ref.shape[0] // cr):
        sl = slice(ci * cr, (ci + 1) * cr)
        y = jax.lax.dot_general(
            ctx_ref[sl, :], wo_ref[...], (((1,), (0,)), ((), ())),
            preferred_element_type=_F32)
        y = y + bo_ref[...] + x_ref[sl, :]
        attn = _ln(y, g1_ref[...], b1_ref[...])
        attn_ref[sl, :] = attn
        ln2 = _ln(attn, g2_ref[...], b2_ref[...])
        ln2_ref[sl, :] = ln2.astype(_BF)
        psums.append(jnp.sum(ln2, axis=0))
    psum = sum(psums)[None, None, :]

    @pl.when(i % nblk_per_b == 0)
    def _init():
        sum_ref[...] = psum

    @pl.when(i % nblk_per_b != 0)
    def _acc():
        sum_ref[...] += psum


def _post_attn(ctx_bf, wo_bf, bo, x2, g1, b1, g2, b2):
    rows = B * S
    br = 512
    nblk_per_b = S // br
    return pl.pallas_call(
        functools.partial(_post_body, nblk_per_b),
        grid=(rows // br,),
        in_specs=[
            pl.BlockSpec((br, D), lambda i: (i, 0)),
            pl.BlockSpec((D, D), lambda i: (0, 0)),
            pl.BlockSpec((1, D), lambda i: (0, 0)),
            pl.BlockSpec((br, D), lambda i: (i, 0)),
            pl.BlockSpec((1, D), lambda i: (0, 0)),
            pl.BlockSpec((1, D), lambda i: (0, 0)),
            pl.BlockSpec((1, D), lambda i: (0, 0)),
            pl.BlockSpec((1, D), lambda i: (0, 0)),
        ],
        out_specs=[
            pl.BlockSpec((br, D), lambda i: (i, 0)),
            pl.BlockSpec((br, D), lambda i: (i, 0)),
            pl.BlockSpec((1, 1, D), lambda i: (i // nblk_per_b, 0, 0)),
        ],
        out_shape=[
            jax.ShapeDtypeStruct((rows, D), _F32),
            jax.ShapeDtypeStruct((rows, D), _BF),
            jax.ShapeDtypeStruct((B, 1, D), _F32),
        ],
        compiler_params=pltpu.CompilerParams(
            dimension_semantics=("arbitrary",)),
    )(ctx_bf, wo_bf, bo, x2, g1, b1, g2, b2)


def _gate_body(sum_ref, gw_ref, logit_ref):
    logit_ref[...] = jax.lax.dot_general(
        sum_ref[...] * (1.0 / S), gw_ref[...], (((1,), (0,)), ((), ())),
        preferred_element_type=_F32,
        precision=jax.lax.Precision.HIGHEST)


def _gate_logits(sum_ln, gate_w):
    return pl.pallas_call(
        _gate_body,
        in_specs=[
            pl.BlockSpec((B, D), lambda: (0, 0)),
            pl.BlockSpec((D, E), lambda: (0, 0)),
        ],
        out_specs=pl.BlockSpec((B, E), lambda: (0, 0)),
        out_shape=jax.ShapeDtypeStruct((B, E), _F32),
    )(sum_ln, gate_w)


def _gelu_exact(x):
    return 0.5 * x * (1.0 + jax.lax.erf(x * 0.7071067811865476))


def _ffn_body(nf, choice_ref, ln_ref, wu_ref, bu_ref, wd_ref, bd_ref,
              out_ref):
    f = pl.program_id(2)
    wu = wu_ref[0].astype(_BF)
    wd = wd_ref[0].astype(_BF)
    cr = 256
    nc = out_ref.shape[1] // cr
    os = []
    for ci in range(nc):
        sl = slice(ci * cr, (ci + 1) * cr)
        xb = ln_ref[0, sl, :]
        h = jax.lax.dot_general(
            xb, wu, (((1,), (0,)), ((), ())),
            preferred_element_type=_F32) + bu_ref[0]
        h = _gelu_exact(h.astype(_BF))
        os.append(jax.lax.dot_general(
            h, wd, (((1,), (0,)), ((), ())),
            preferred_element_type=_F32))

    for ci in range(nc):
        sl = slice(ci * cr, (ci + 1) * cr)
        o = os[ci]

        @pl.when(f == 0)
        def _init(sl=sl, o=o):
            out_ref[0, sl, :] = o

        @pl.when((f > 0) & (f < nf - 1))
        def _acc(sl=sl, o=o):
            out_ref[0, sl, :] = out_ref[0, sl, :] + o

        @pl.when(f == nf - 1)
        def _fini(sl=sl, o=o):
            out_ref[0, sl, :] = out_ref[0, sl, :] + o + bd_ref[0]


def _moe_ffn(choice, ln3, w_up, b_up3, w_down, b_down3):
    bs = 1024
    fb = 2048
    nf = DFF // fb
    grid = (B, S // bs, nf)
    return pl.pallas_call(
        functools.partial(_ffn_body, nf),
        grid_spec=pltpu.PrefetchScalarGridSpec(
            num_scalar_prefetch=1,
            grid=grid,
            in_specs=[
                pl.BlockSpec((1, bs, D), lambda b, s, f, c: (b, s, 0)),
                pl.BlockSpec((1, D, fb), lambda b, s, f, c: (c[b], 0, f)),
                pl.BlockSpec((1, 1, fb), lambda b, s, f, c: (c[b], 0, f)),
                pl.BlockSpec((1, fb, D), lambda b, s, f, c: (c[b], f, 0)),
                pl.BlockSpec((1, 1, D), lambda b, s, f, c: (c[b], 0, 0)),
            ],
            out_specs=pl.BlockSpec((1, bs, D), lambda b, s, f, c: (b, s, 0)),
        ),
        out_shape=jax.ShapeDtypeStruct((B, S, D), _F32),
        compiler_params=pltpu.CompilerParams(
            dimension_semantics=("arbitrary", "arbitrary", "arbitrary")),
    )(choice, ln3, w_up, b_up3, w_down, b_down3)


def kernel(hidden_states, Wq, bq, Wk, bk, Wv, bv, Wo, bo,
           ln_attn_g, ln_attn_b, ln_moe_g, ln_moe_b,
           gate_W, W_up, b_up, W_down, b_down):
    x2 = hidden_states.reshape(B * S, D)
    x_bf = x2.astype(_BF)

    wqkv = jnp.concatenate([Wq * 0.125, Wk, Wv], axis=1).astype(_BF)
    bqkv = jnp.concatenate([bq * 0.125, bk, bv])[None, :]
    qkv = _qkv_proj(x_bf, wqkv, bqkv)

    qkv4 = qkv.reshape(B, S, 3, H, DH)
    q = qkv4[:, :, 0].transpose(0, 2, 1, 3)
    kt = qkv4[:, :, 1].transpose(0, 2, 3, 1)
    v = qkv4[:, :, 2].transpose(0, 2, 1, 3)
    ones = jnp.ones((B, H, S, 1), _BF)
    v_aug = jnp.concatenate([v, jnp.broadcast_to(ones, (B, H, S, DH))],
                            axis=-1)
    ctx = _attention(q, kt, v_aug)
    ctx2 = ctx.transpose(0, 2, 1, 3).reshape(B * S, D)

    attn2, ln2, sum_ln = _post_attn(
        ctx2, Wo.astype(_BF), bo[None, :], x2,
        ln_attn_g[None, :], ln_attn_b[None, :],
        ln_moe_g[None, :], ln_moe_b[None, :])
    router_logits = _gate_logits(sum_ln.reshape(B, D), gate_W)
    choice = jnp.argmax(router_logits, axis=-1).astype(jnp.int32)

    moe = _moe_ffn(choice,
                   ln2.reshape(B, S, D),
                   W_up, b_up.reshape(E, 1, DFF),
                   W_down, b_down.reshape(E, 1, D))
    out = moe + attn2.reshape(B, S, D)
    return (out, router_logits)


# speedup vs baseline: 1.1622x; 1.0001x over previous
import functools

import jax
import jax.numpy as jnp
from jax.experimental import pallas as pl
from jax.experimental.pallas import tpu as pltpu

B, S, D, H, DFF, E = 4, 2048, 1024, 16, 4096, 8
DH = D // H
EPS = 1e-12

_BF = jnp.bfloat16
_F32 = jnp.float32


def _qkv_body(x_ref, w_ref, b_ref, o_ref):
    cr = 256
    nc = o_ref.shape[0] // cr
    accs = []
    for ci in range(nc):
        sl = slice(ci * cr, (ci + 1) * cr)
        accs.append(jax.lax.dot_general(
            x_ref[sl, :], w_ref[...], (((1,), (0,)), ((), ())),
            preferred_element_type=_F32))
    for ci in range(nc):
        sl = slice(ci * cr, (ci + 1) * cr)
        o_ref[sl, :] = (accs[ci] + b_ref[...]).astype(_BF)


def _qkv_proj(x_bf, w_bf, bias):
    rows = B * S
    br = 512
    return pl.pallas_call(
        _qkv_body,
        grid=(rows // br,),
        in_specs=[
            pl.BlockSpec((br, D), lambda i: (i, 0)),
            pl.BlockSpec((D, 3 * D), lambda i: (0, 0)),
            pl.BlockSpec((1, 3 * D), lambda i: (0, 0)),
        ],
        out_specs=pl.BlockSpec((br, 3 * D), lambda i: (i, 0)),
        out_shape=jax.ShapeDtypeStruct((rows, 3 * D), _BF),
    )(x_bf, w_bf, bias)


def _attn_body(q_ref, kt_ref, v_ref, o_ref):
    kt = kt_ref[0, 0]
    v = v_ref[0, 0]
    cr = 512
    nc = o_ref.shape[2] // cr
    es = []
    for ci in range(nc):
        q = q_ref[0, 0, ci * cr:(ci + 1) * cr, :]
        s = jax.lax.dot_general(
            q, kt, (((1,), (0,)), ((), ())),
            preferred_element_type=_F32).astype(_BF)
        m = jnp.max(s, axis=1, keepdims=True)
        es.append(jnp.exp(s - m))
    for ci in range(nc):
        ctx = jax.lax.dot_general(
            es[ci], v, (((1,), (0,)), ((), ())),
            preferred_element_type=_F32)
        l = ctx[:, DH:DH + 1]
        o_ref[0, 0, ci * cr:(ci + 1) * cr, :] = (
            ctx[:, :DH] * (1.0 / l)).astype(_BF)


def _attention(q, kt, v_aug):
    bq = 2048
    return pl.pallas_call(
        _attn_body,
        grid=(B, H, S // bq),
        in_specs=[
            pl.BlockSpec((1, 1, bq, DH), lambda b, h, i: (b, h, i, 0)),
            pl.BlockSpec((1, 1, DH, S), lambda b, h, i: (b, h, 0, 0)),
            pl.BlockSpec((1, 1, S, 2 * DH), lambda b, h, i: (b, h, 0, 0)),
        ],
        out_specs=pl.BlockSpec((1, 1, bq, DH), lambda b, h, i: (b, h, i, 0)),
        out_shape=jax.ShapeDtypeStruct((B, H, S, DH), _BF),
        compiler_params=pltpu.CompilerParams(
            dimension_semantics=("parallel", "parallel", "parallel")),
    )(q, kt, v_aug)


def _ln(y, g, b):
    mu = jnp.mean(y, axis=1, keepdims=True)
    yc = y - mu
    var = jnp.mean(yc * yc, axis=1, keepdims=True)
    return yc * jax.lax.rsqrt(var + EPS) * g + b


def _post_body(nblk_per_b, ctx_ref, wo_ref, bo_ref, x_ref,
               g1_ref, b1_ref, g2_ref, b2_ref,
               attn_ref, ln2_ref, sum_ref):
    i = pl.program_id(0)
    cr = 256
    psums = []
    for ci in range(ctx_ref.shape[0] // cr):
        sl = slice(ci * cr, (ci + 1) * cr)
        y = jax.lax.dot_general(
            ctx_ref[sl, :], wo_ref[...], (((1,), (0,)), ((), ())),
            preferred_element_type=_F32)
        y = y + bo_ref[...] + x_ref[sl, :]
        attn = _ln(y, g1_ref[...], b1_ref[...])
        attn_ref[sl, :] = attn
        ln2 = _ln(attn, g2_ref[...], b2_ref[...])
        ln2_ref[sl, :] = ln2.astype(_BF)
        psums.append(jnp.sum(ln2, axis=0))
    psum = sum(psums)[None, None, :]

    @pl.when(i % nblk_per_b == 0)
    def _init():
        sum_ref[...] = psum

    @pl.when(i % nblk_per_b != 0)
    def _acc():
        sum_ref[...] += psum


def _post_attn(ctx_bf, wo_bf, bo, x2, g1, b1, g2, b2):
    rows = B * S
    br = 512
    nblk_per_b = S // br
    return pl.pallas_call(
        functools.partial(_post_body, nblk_per_b),
        grid=(rows // br,),
        in_specs=[
            pl.BlockSpec((br, D), lambda i: (i, 0)),
            pl.BlockSpec((D, D), lambda i: (0, 0)),
            pl.BlockSpec((1, D), lambda i: (0, 0)),
            pl.BlockSpec((br, D), lambda i: (i, 0)),
            pl.BlockSpec((1, D), lambda i: (0, 0)),
            pl.BlockSpec((1, D), lambda i: (0, 0)),
            pl.BlockSpec((1, D), lambda i: (0, 0)),
            pl.BlockSpec((1, D), lambda i: (0, 0)),
        ],
        out_specs=[
            pl.BlockSpec((br, D), lambda i: (i, 0)),
            pl.BlockSpec((br, D), lambda i: (i, 0)),
            pl.BlockSpec((1, 1, D), lambda i: (i // nblk_per_b, 0, 0)),
        ],
        out_shape=[
            jax.ShapeDtypeStruct((rows, D), _F32),
            jax.ShapeDtypeStruct((rows, D), _BF),
            jax.ShapeDtypeStruct((B, 1, D), _F32),
        ],
        compiler_params=pltpu.CompilerParams(
            dimension_semantics=("arbitrary",)),
    )(ctx_bf, wo_bf, bo, x2, g1, b1, g2, b2)


def _gate_body(sum_ref, gw_ref, logit_ref):
    logit_ref[...] = jax.lax.dot_general(
        sum_ref[...] * (1.0 / S), gw_ref[...], (((1,), (0,)), ((), ())),
        preferred_element_type=_F32,
        precision=jax.lax.Precision.HIGHEST)


def _gate_logits(sum_ln, gate_w):
    return pl.pallas_call(
        _gate_body,
        in_specs=[
            pl.BlockSpec((B, D), lambda: (0, 0)),
            pl.BlockSpec((D, E), lambda: (0, 0)),
        ],
        out_specs=pl.BlockSpec((B, E), lambda: (0, 0)),
        out_shape=jax.ShapeDtypeStruct((B, E), _F32),
    )(sum_ln, gate_w)


def _gelu_exact(x):
    return 0.5 * x * (1.0 + jax.lax.erf(x * 0.7071067811865476))


def _ffn_body(nf, choice_ref, ln_ref, wu_ref, bu_ref, wd_ref, bd_ref,
              out_ref):
    f = pl.program_id(2)
    wu = wu_ref[0].astype(_BF)
    wd = wd_ref[0].astype(_BF)
    cr = 256
    nc = out_ref.shape[1] // cr
    hs = []
    for ci in range(nc):
        sl = slice(ci * cr, (ci + 1) * cr)
        xb = ln_ref[0, sl, :]
        h = jax.lax.dot_general(
            xb, wu, (((1,), (0,)), ((), ())),
            preferred_element_type=_F32) + bu_ref[0]
        hs.append(_gelu_exact(h.astype(_BF)))
    os = []
    for ci in range(nc):
        os.append(jax.lax.dot_general(
            hs[ci], wd, (((1,), (0,)), ((), ())),
            preferred_element_type=_F32))

    for ci in range(nc):
        sl = slice(ci * cr, (ci + 1) * cr)
        o = os[ci]

        @pl.when(f == 0)
        def _init(sl=sl, o=o):
            out_ref[0, sl, :] = o

        @pl.when((f > 0) & (f < nf - 1))
        def _acc(sl=sl, o=o):
            out_ref[0, sl, :] = out_ref[0, sl, :] + o

        @pl.when(f == nf - 1)
        def _fini(sl=sl, o=o):
            out_ref[0, sl, :] = out_ref[0, sl, :] + o + bd_ref[0]


def _moe_ffn(choice, ln3, w_up, b_up3, w_down, b_down3):
    bs = 1024
    fb = 2048
    nf = DFF // fb
    grid = (B, S // bs, nf)
    return pl.pallas_call(
        functools.partial(_ffn_body, nf),
        grid_spec=pltpu.PrefetchScalarGridSpec(
            num_scalar_prefetch=1,
            grid=grid,
            in_specs=[
                pl.BlockSpec((1, bs, D), lambda b, s, f, c: (b, s, 0)),
                pl.BlockSpec((1, D, fb), lambda b, s, f, c: (c[b], 0, f)),
                pl.BlockSpec((1, 1, fb), lambda b, s, f, c: (c[b], 0, f)),
                pl.BlockSpec((1, fb, D), lambda b, s, f, c: (c[b], f, 0)),
                pl.BlockSpec((1, 1, D), lambda b, s, f, c: (c[b], 0, 0)),
            ],
            out_specs=pl.BlockSpec((1, bs, D), lambda b, s, f, c: (b, s, 0)),
        ),
        out_shape=jax.ShapeDtypeStruct((B, S, D), _F32),
        compiler_params=pltpu.CompilerParams(
            dimension_semantics=("arbitrary", "arbitrary", "arbitrary")),
    )(choice, ln3, w_up, b_up3, w_down, b_down3)


def kernel(hidden_states, Wq, bq, Wk, bk, Wv, bv, Wo, bo,
           ln_attn_g, ln_attn_b, ln_moe_g, ln_moe_b,
           gate_W, W_up, b_up, W_down, b_down):
    x2 = hidden_states.reshape(B * S, D)
    x_bf = x2.astype(_BF)

    wqkv = jnp.concatenate([Wq * 0.125, Wk, Wv], axis=1).astype(_BF)
    bqkv = jnp.concatenate([bq * 0.125, bk, bv])[None, :]
    qkv = _qkv_proj(x_bf, wqkv, bqkv)

    qkv4 = qkv.reshape(B, S, 3, H, DH)
    q = qkv4[:, :, 0].transpose(0, 2, 1, 3)
    kt = qkv4[:, :, 1].transpose(0, 2, 3, 1)
    v = qkv4[:, :, 2].transpose(0, 2, 1, 3)
    ones = jnp.ones((B, H, S, 1), _BF)
    v_aug = jnp.concatenate([v, jnp.broadcast_to(ones, (B, H, S, DH))],
                            axis=-1)
    ctx = _attention(q, kt, v_aug)
    ctx2 = ctx.transpose(0, 2, 1, 3).reshape(B * S, D)

    attn2, ln2, sum_ln = _post_attn(
        ctx2, Wo.astype(_BF), bo[None, :], x2,
        ln_attn_g[None, :], ln_attn_b[None, :],
        ln_moe_g[None, :], ln_moe_b[None, :])
    router_logits = _gate_logits(sum_ln.reshape(B, D), gate_W)
    choice = jnp.argmax(router_logits, axis=-1).astype(jnp.int32)

    moe = _moe_ffn(choice,
                   ln2.reshape(B, S, D),
                   W_up, b_up.reshape(E, 1, DFF),
                   W_down, b_down.reshape(E, 1, D))
    out = moe + attn2.reshape(B, S, D)
    return (out, router_logits)
